# baseline (speedup 1.0000x reference)
import functools

import jax
import jax.numpy as jnp
import numpy as np
from jax import lax
from jax.experimental import pallas as pl
from jax.experimental.pallas import tpu as pltpu
from jax.experimental.pallas import tpu_sc as plsc

_N = 10000
_E = 320000
_F = 128

_NTILE = 16
_NCORE = 2
_NW = _NTILE * _NCORE

_K = 120
_B = 3
_NCHUNK = 2688
_EPAD = _NCHUNK * _K
_CHF = 156
_CHS = 12
_FAST_CORE = 0

_CHD = 84
_BD = 3

_NPAD = 10112
_RPT = _NPAD // _NTILE

_R = 632
_NBLK = _NPAD // _R

_BN_S = float(1.0 / np.sqrt(1.0 + 1e-4))

_sc_mesh = plsc.VectorSubcoreMesh(core_axis_name="c", subcore_axis_name="s")


def _zero_fill(buf, val=0.0):
    if buf.dtype == jnp.bfloat16:
        v = jnp.full((2, 16), val, jnp.bfloat16)

        @pl.loop(0, _K // 2)
        def _fill(i):
            r = pl.multiple_of(i * 2, 2)
            for j in range(_F // 16):
                buf[pl.ds(r, 2), pl.ds(j * 16, 16)] = v
    else:
        v = jnp.full((16,), val, jnp.float32)

        @pl.loop(0, _K)
        def _fill(r):
            for j in range(_F // 16):
                buf[r, pl.ds(j * 16, 16)] = v


def _init_acc(acc, zrows, r0):
    nfull = _RPT // _K
    for t in range(nfull):
        pltpu.sync_copy(zrows, acc.at[pl.ds(r0 + t * _K, _K)])
    rem = _RPT - nfull * _K
    if rem:
        pltpu.sync_copy(zrows.at[pl.ds(0, rem)],
                        acc.at[pl.ds(r0 + nfull * _K, rem)])


@functools.partial(
    pl.kernel,
    out_type=jax.ShapeDtypeStruct((_NCORE, _NPAD, _F), jnp.float32),
    mesh=_sc_mesh,
    scratch_types=[
        pltpu.VMEM_SHARED((_NPAD, _F), jnp.float32),
        pltpu.VMEM((_K, _F), jnp.float32),
        pltpu.VMEM((_CHD, _K), jnp.int32),
        pltpu.SemaphoreType.DMA,
    ],
)
def _degree(dst_hbm, out_hbm, acc, rows, didx, ssem):
    c = lax.axis_index("c")
    s = lax.axis_index("s")
    wid = s * _NCORE + c
    r0 = s * _RPT

    _zero_fill(rows, 0.0)
    _init_acc(acc, rows, r0)
    _zero_fill(rows, 1.0)
    pltpu.sync_copy(dst_hbm.at[wid], didx)
    plsc.subcore_barrier()

    @pl.loop(0, _CHD // _BD)
    def _grp(g):
        c0 = g * _BD
        for b in range(_BD):
            pltpu.async_copy(rows, acc.at[didx.at[c0 + b]], ssem, add=True)
        for b in range(_BD):
            pltpu.make_async_copy(rows, acc.at[didx.at[c0 + b]], ssem).wait()

    plsc.subcore_barrier()
    pltpu.sync_copy(acc.at[pl.ds(r0, _RPT)], out_hbm.at[c, pl.ds(r0, _RPT)])


@functools.partial(
    pl.kernel,
    out_type=jax.ShapeDtypeStruct((_NCORE, _NPAD, _F), jnp.float32),
    mesh=_sc_mesh,
    scratch_types=[
        pltpu.VMEM_SHARED((_NPAD, _F), jnp.float32),
        [pltpu.VMEM((_K, _F), jnp.float32) for _ in range(_B)],
        [pltpu.VMEM((2, _K), jnp.int32) for _ in range(_B)],
        [pltpu.VMEM((2, _K), jnp.int32) for _ in range(_B)],
        [pltpu.SemaphoreType.DMA for _ in range(_B)],
        [pltpu.SemaphoreType.DMA for _ in range(_B)],
        [pltpu.SemaphoreType.DMA for _ in range(_B)],
    ],
)
def _propagate(g_hbm, pair_hbm, out_hbm, acc, rows, idx0, idx1,
               gsem, ssem, isem):
    c = lax.axis_index("c")
    s = lax.axis_index("s")
    r0 = s * _RPT

    is_fast = c == _FAST_CORE
    nch = jnp.where(is_fast, _CHF, _CHS)
    ngrp = nch // _B
    base = jnp.where(is_fast, s * _CHF, _NTILE * _CHF + s * _CHS)

    _zero_fill(rows[0], 0.0)
    _init_acc(acc, rows[0], r0)

    for b in range(_B):
        pltpu.sync_copy(pair_hbm.at[base + b], idx0[b])
    plsc.subcore_barrier()

    for b in range(_B):
        pltpu.async_copy(g_hbm.at[idx0[b].at[0]], rows[b], gsem[b])

    def do_group(g, cur, nxt):
        c0 = g * _B
        for b in range(_B):
            @pl.when(g < ngrp - 1)
            def _pf():
                pltpu.async_copy(pair_hbm.at[base + c0 + _B + b], nxt[b],
                                 isem[b])
        for b in range(_B):
            pltpu.make_async_copy(g_hbm.at[cur[b].at[0]], rows[b],
                                  gsem[b]).wait()
            pltpu.async_copy(rows[b], acc.at[cur[b].at[1]], ssem[b],
                             add=True)
        for b in range(_B):
            pltpu.make_async_copy(rows[b], acc.at[cur[b].at[1]],
                                  ssem[b]).wait()

            @pl.when(g < ngrp - 1)
            def _rf():
                pltpu.make_async_copy(pair_hbm.at[base + c0 + _B + b], nxt[b],
                                      isem[b]).wait()
                pltpu.async_copy(g_hbm.at[nxt[b].at[0]], rows[b], gsem[b])

    @pl.loop(0, ngrp // 2)
    def _grp2(i):
        do_group(2 * i, idx0, idx1)
        do_group(2 * i + 1, idx1, idx0)

    plsc.subcore_barrier()
    pltpu.sync_copy(acc.at[pl.ds(r0, _RPT)], out_hbm.at[c, pl.ds(r0, _RPT)])


def _dinv_block(degp_ref):
    deg = degp_ref[0, :, 0:1] + degp_ref[1, :, 0:1] + 1.0
    return lax.rsqrt(deg)


def _tc_first_body(x_ref, degp_ref, w_ref, h_out, g_out):
    dinv = _dinv_block(degp_ref)
    h = jnp.dot(x_ref[...], w_ref[...], preferred_element_type=jnp.float32,
                precision=lax.Precision.HIGHEST)
    h_out[...] = h
    g_out[...] = h * dinv


_DEGP_SPEC = pl.BlockSpec((_NCORE, _R, 16), lambda i: (0, i, 0))
_P_SPEC = pl.BlockSpec((_NCORE, _R, _F), lambda i: (0, i, 0))
_ROW_SPEC = pl.BlockSpec((_R, _F), lambda i: (i, 0))
_VEC_SPEC = pl.BlockSpec((1, _F), lambda i: (0, 0))
_W_SPEC = pl.BlockSpec((_F, _F), lambda i: (0, 0))

_tc_first = pl.pallas_call(
    _tc_first_body,
    grid=(_NBLK,),
    in_specs=[_ROW_SPEC, _DEGP_SPEC, _W_SPEC],
    out_specs=[_ROW_SPEC, _ROW_SPEC],
    out_shape=[
        jax.ShapeDtypeStruct((_NPAD, _F), jnp.float32),
        jax.ShapeDtypeStruct((_NPAD, _F), jnp.float32),
    ],
)


def _make_tc_layer(act, next_src):
    def body(*refs):
        it = iter(refs)
        degp_ref = next(it)
        p_ref = next(it)
        h_ref = next(it)
        b_ref = next(it)
        gam_ref = next(it)
        bet_ref = next(it)
        w_ref = next(it) if next_src else None
        ext_ref = next(it) if next_src == "extra" else None
        a_out = next(it)
        if next_src:
            h_out = next(it)
            g_out = next(it)

        dinv = _dinv_block(degp_ref)
        accv = (p_ref[0].astype(jnp.float32) + p_ref[1].astype(jnp.float32))
        h = h_ref[...]
        out = dinv * accv + (dinv * dinv) * h + b_ref[...]
        if act == "sig":
            z = jax.nn.sigmoid(out)
        elif act == "sigh":
            z = jax.nn.sigmoid(out * 0.5)
        else:
            z = out
        a = z * (gam_ref[...] * _BN_S) + bet_ref[...]
        a_out[...] = a
        if next_src:
            src = ext_ref[...] if next_src == "extra" else a
            hn = jnp.dot(src, w_ref[...], preferred_element_type=jnp.float32,
                         precision=lax.Precision.HIGHEST)
            h_out[...] = hn
            g_out[...] = hn * dinv

    in_specs = [_DEGP_SPEC, _P_SPEC, _ROW_SPEC, _VEC_SPEC, _VEC_SPEC, _VEC_SPEC]
    n_out = 1
    if next_src:
        in_specs.append(_W_SPEC)
        n_out = 3
    if next_src == "extra":
        in_specs.append(_ROW_SPEC)

    out_shape = [jax.ShapeDtypeStruct((_NPAD, _F), jnp.float32)] * n_out
    return pl.pallas_call(
        body,
        grid=(_NBLK,),
        in_specs=in_specs,
        out_specs=[_ROW_SPEC] * n_out,
        out_shape=out_shape,
    )


_tc_sig_epi = _make_tc_layer("sig", "epi")
_tc_id_extra = _make_tc_layer("id", "extra")
_tc_sigh_none = _make_tc_layer("sigh", None)


def kernel(x, edge_index, batch, params):
    del batch
    src = edge_index[0]
    dst = edge_index[1]
    pad_e = _EPAD - _E
    fill = jnp.full((pad_e,), _N, jnp.int32)
    srcp = jnp.concatenate([src, fill]).reshape(_NCHUNK, _K)
    dstp = jnp.concatenate([dst, fill]).reshape(_NCHUNK, _K)
    pair = jnp.stack([srcp, dstp], axis=1)
    dst_deg = dstp.reshape(_NW, _CHD, _K)
    xp = jnp.concatenate([x, jnp.zeros((_NPAD - _N, _F), x.dtype)], axis=0)

    def vec(p, name):
        return p[name].reshape(1, _F)

    degp = _degree(dst_deg)[:, :, :16]

    h, g = _tc_first(xp, degp, params["enc0"]["W"])

    p = params["enc0"]
    P = _propagate(g, pair)
    _, h, g = _tc_sig_epi(degp, P, h, vec(p, "b"), vec(p, "gamma"),
                          vec(p, "beta"), params["enc1"]["W"])

    p = params["enc1"]
    P = _propagate(g, pair)
    x_enc, h, g = _tc_sig_epi(degp, P, h, vec(p, "b"), vec(p, "gamma"),
                              vec(p, "beta"), params["attr0"]["W"])

    p = params["attr0"]
    P = _propagate(g, pair)
    _, h, g = _tc_sig_epi(degp, P, h, vec(p, "b"), vec(p, "gamma"),
                          vec(p, "beta"), params["attr1"]["W"])

    p = params["attr1"]
    P = _propagate(g, pair)
    x_att, h, g = _tc_id_extra(degp, P, h, vec(p, "b"), vec(p, "gamma"),
                               vec(p, "beta"), params["str0"]["W"], x_enc)

    p = params["str0"]
    P = _propagate(g, pair)
    _, h, g = _tc_sig_epi(degp, P, h, vec(p, "b"), vec(p, "gamma"),
                          vec(p, "beta"), params["str1"]["W"])

    p = params["str1"]
    P = _propagate(g, pair)
    (x_str,) = _tc_sigh_none(degp, P, h, vec(p, "b"), vec(p, "gamma"),
                             vec(p, "beta"))

    return (x_str[:_N], x_att[:_N], x_enc[:_N])


# speedup vs baseline: 1.0902x; 1.0902x over previous
import functools

import jax
import jax.numpy as jnp
import numpy as np
from jax import lax
from jax.experimental import pallas as pl
from jax.experimental.pallas import tpu as pltpu
from jax.experimental.pallas import tpu_sc as plsc

_N = 10000
_E = 320000
_F = 128

_NTILE = 16
_NCORE = 2
_NW = _NTILE * _NCORE

_K = 120
_B = 3
_NCHUNK = 2688
_EPAD = _NCHUNK * _K
_CHF = 138
_CHS = 30
_FAST_CORE = 0

_CHD = 84
_BD = 3

_NPAD = 10112
_RPT = _NPAD // _NTILE

_R = 632
_NBLK = _NPAD // _R

_BN_S = float(1.0 / np.sqrt(1.0 + 1e-4))

_sc_mesh = plsc.VectorSubcoreMesh(core_axis_name="c", subcore_axis_name="s")


def _zero_fill(buf, val=0.0):
    if buf.dtype == jnp.bfloat16:
        v = jnp.full((2, 16), val, jnp.bfloat16)

        @pl.loop(0, _K // 2)
        def _fill(i):
            r = pl.multiple_of(i * 2, 2)
            for j in range(_F // 16):
                buf[pl.ds(r, 2), pl.ds(j * 16, 16)] = v
    else:
        v = jnp.full((16,), val, jnp.float32)

        @pl.loop(0, _K)
        def _fill(r):
            for j in range(_F // 16):
                buf[r, pl.ds(j * 16, 16)] = v


def _init_acc(acc, zrows, r0):
    nfull = _RPT // _K
    for t in range(nfull):
        pltpu.sync_copy(zrows, acc.at[pl.ds(r0 + t * _K, _K)])
    rem = _RPT - nfull * _K
    if rem:
        pltpu.sync_copy(zrows.at[pl.ds(0, rem)],
                        acc.at[pl.ds(r0 + nfull * _K, rem)])


@functools.partial(
    pl.kernel,
    out_type=jax.ShapeDtypeStruct((_NCORE, _NPAD, _F), jnp.float32),
    mesh=_sc_mesh,
    scratch_types=[
        pltpu.VMEM_SHARED((_NPAD, _F), jnp.float32),
        pltpu.VMEM((_K, _F), jnp.float32),
        pltpu.VMEM((_CHD, _K), jnp.int32),
        pltpu.SemaphoreType.DMA,
    ],
)
def _degree(dst_hbm, out_hbm, acc, rows, didx, ssem):
    c = lax.axis_index("c")
    s = lax.axis_index("s")
    wid = s * _NCORE + c
    r0 = s * _RPT

    _zero_fill(rows, 0.0)
    _init_acc(acc, rows, r0)
    _zero_fill(rows, 1.0)
    pltpu.sync_copy(dst_hbm.at[wid], didx)
    plsc.subcore_barrier()

    @pl.loop(0, _CHD // _BD)
    def _grp(g):
        c0 = g * _BD
        for b in range(_BD):
            pltpu.async_copy(rows, acc.at[didx.at[c0 + b]], ssem, add=True)
        for b in range(_BD):
            pltpu.make_async_copy(rows, acc.at[didx.at[c0 + b]], ssem).wait()

    plsc.subcore_barrier()
    pltpu.sync_copy(acc.at[pl.ds(r0, _RPT)], out_hbm.at[c, pl.ds(r0, _RPT)])


@functools.partial(
    pl.kernel,
    out_type=jax.ShapeDtypeStruct((_NCORE, _NPAD, _F), jnp.float32),
    mesh=_sc_mesh,
    scratch_types=[
        pltpu.VMEM_SHARED((_NPAD, _F), jnp.float32),
        [pltpu.VMEM((_K, _F), jnp.float32) for _ in range(_B)],
        [pltpu.VMEM((2, _K), jnp.int32) for _ in range(_B)],
        [pltpu.VMEM((2, _K), jnp.int32) for _ in range(_B)],
        [pltpu.SemaphoreType.DMA for _ in range(_B)],
        [pltpu.SemaphoreType.DMA for _ in range(_B)],
        [pltpu.SemaphoreType.DMA for _ in range(_B)],
    ],
)
def _propagate(g_hbm, pair_hbm, out_hbm, acc, rows, idx0, idx1,
               gsem, ssem, isem):
    c = lax.axis_index("c")
    s = lax.axis_index("s")
    r0 = s * _RPT

    is_fast = c == _FAST_CORE
    nch = jnp.where(is_fast, _CHF, _CHS)
    ngrp = nch // _B
    base = jnp.where(is_fast, s * _CHF, _NTILE * _CHF + s * _CHS)

    _zero_fill(rows[0], 0.0)
    _init_acc(acc, rows[0], r0)

    for b in range(_B):
        pltpu.sync_copy(pair_hbm.at[base + b], idx0[b])
    plsc.subcore_barrier()

    for b in range(_B):
        pltpu.async_copy(g_hbm.at[idx0[b].at[0]], rows[b], gsem[b])

    def do_group(g, cur, nxt):
        c0 = g * _B
        for b in range(_B):
            @pl.when(g < ngrp - 1)
            def _pf():
                pltpu.async_copy(pair_hbm.at[base + c0 + _B + b], nxt[b],
                                 isem[b])
        for b in range(_B):
            pltpu.make_async_copy(g_hbm.at[cur[b].at[0]], rows[b],
                                  gsem[b]).wait()
            pltpu.async_copy(rows[b], acc.at[cur[b].at[1]], ssem[b],
                             add=True)
        for b in range(_B):
            pltpu.make_async_copy(rows[b], acc.at[cur[b].at[1]],
                                  ssem[b]).wait()

            @pl.when(g < ngrp - 1)
            def _rf():
                pltpu.make_async_copy(pair_hbm.at[base + c0 + _B + b], nxt[b],
                                      isem[b]).wait()
                pltpu.async_copy(g_hbm.at[nxt[b].at[0]], rows[b], gsem[b])

    @pl.loop(0, ngrp // 2)
    def _grp2(i):
        do_group(2 * i, idx0, idx1)
        do_group(2 * i + 1, idx1, idx0)

    plsc.subcore_barrier()
    pltpu.sync_copy(acc.at[pl.ds(r0, _RPT)], out_hbm.at[c, pl.ds(r0, _RPT)])


def _dinv_block(degp_ref):
    deg = degp_ref[0, :, 0:1] + degp_ref[1, :, 0:1] + 1.0
    return lax.rsqrt(deg)


def _tc_first_body(x_ref, degp_ref, w_ref, h_out, g_out):
    dinv = _dinv_block(degp_ref)
    h = jnp.dot(x_ref[...], w_ref[...], preferred_element_type=jnp.float32,
                precision=lax.Precision.HIGHEST)
    h_out[...] = h
    g_out[...] = h * dinv


_DEGP_SPEC = pl.BlockSpec((_NCORE, _R, 16), lambda i: (0, i, 0))
_P_SPEC = pl.BlockSpec((_NCORE, _R, _F), lambda i: (0, i, 0))
_ROW_SPEC = pl.BlockSpec((_R, _F), lambda i: (i, 0))
_VEC_SPEC = pl.BlockSpec((1, _F), lambda i: (0, 0))
_W_SPEC = pl.BlockSpec((_F, _F), lambda i: (0, 0))

_tc_first = pl.pallas_call(
    _tc_first_body,
    grid=(_NBLK,),
    in_specs=[_ROW_SPEC, _DEGP_SPEC, _W_SPEC],
    out_specs=[_ROW_SPEC, _ROW_SPEC],
    out_shape=[
        jax.ShapeDtypeStruct((_NPAD, _F), jnp.float32),
        jax.ShapeDtypeStruct((_NPAD, _F), jnp.float32),
    ],
)


def _make_tc_layer(act, next_src):
    def body(*refs):
        it = iter(refs)
        degp_ref = next(it)
        p_ref = next(it)
        h_ref = next(it)
        b_ref = next(it)
        gam_ref = next(it)
        bet_ref = next(it)
        w_ref = next(it) if next_src else None
        ext_ref = next(it) if next_src == "extra" else None
        a_out = next(it)
        if next_src:
            h_out = next(it)
            g_out = next(it)

        dinv = _dinv_block(degp_ref)
        accv = (p_ref[0].astype(jnp.float32) + p_ref[1].astype(jnp.float32))
        h = h_ref[...]
        out = dinv * accv + (dinv * dinv) * h + b_ref[...]
        if act == "sig":
            z = jax.nn.sigmoid(out)
        elif act == "sigh":
            z = jax.nn.sigmoid(out * 0.5)
        else:
            z = out
        a = z * (gam_ref[...] * _BN_S) + bet_ref[...]
        a_out[...] = a
        if next_src:
            src = ext_ref[...] if next_src == "extra" else a
            hn = jnp.dot(src, w_ref[...], preferred_element_type=jnp.float32,
                         precision=lax.Precision.HIGHEST)
            h_out[...] = hn
            g_out[...] = hn * dinv

    in_specs = [_DEGP_SPEC, _P_SPEC, _ROW_SPEC, _VEC_SPEC, _VEC_SPEC, _VEC_SPEC]
    n_out = 1
    if next_src:
        in_specs.append(_W_SPEC)
        n_out = 3
    if next_src == "extra":
        in_specs.append(_ROW_SPEC)

    out_shape = [jax.ShapeDtypeStruct((_NPAD, _F), jnp.float32)] * n_out
    return pl.pallas_call(
        body,
        grid=(_NBLK,),
        in_specs=in_specs,
        out_specs=[_ROW_SPEC] * n_out,
        out_shape=out_shape,
    )


_tc_sig_epi = _make_tc_layer("sig", "epi")
_tc_id_extra = _make_tc_layer("id", "extra")
_tc_sigh_none = _make_tc_layer("sigh", None)


def kernel(x, edge_index, batch, params):
    del batch
    src = edge_index[0]
    dst = edge_index[1]
    pad_e = _EPAD - _E
    fill = jnp.full((pad_e,), _N, jnp.int32)
    srcp = jnp.concatenate([src, fill]).reshape(_NCHUNK, _K)
    dstp = jnp.concatenate([dst, fill]).reshape(_NCHUNK, _K)
    pair = jnp.stack([srcp, dstp], axis=1)
    dst_deg = dstp.reshape(_NW, _CHD, _K)
    xp = jnp.concatenate([x, jnp.zeros((_NPAD - _N, _F), x.dtype)], axis=0)

    def vec(p, name):
        return p[name].reshape(1, _F)

    degp = _degree(dst_deg)[:, :, :16]

    h, g = _tc_first(xp, degp, params["enc0"]["W"])

    p = params["enc0"]
    P = _propagate(g, pair)
    _, h, g = _tc_sig_epi(degp, P, h, vec(p, "b"), vec(p, "gamma"),
                          vec(p, "beta"), params["enc1"]["W"])

    p = params["enc1"]
    P = _propagate(g, pair)
    x_enc, h, g = _tc_sig_epi(degp, P, h, vec(p, "b"), vec(p, "gamma"),
                              vec(p, "beta"), params["attr0"]["W"])

    p = params["attr0"]
    P = _propagate(g, pair)
    _, h, g = _tc_sig_epi(degp, P, h, vec(p, "b"), vec(p, "gamma"),
                          vec(p, "beta"), params["attr1"]["W"])

    p = params["attr1"]
    P = _propagate(g, pair)
    x_att, h, g = _tc_id_extra(degp, P, h, vec(p, "b"), vec(p, "gamma"),
                               vec(p, "beta"), params["str0"]["W"], x_enc)

    p = params["str0"]
    P = _propagate(g, pair)
    _, h, g = _tc_sig_epi(degp, P, h, vec(p, "b"), vec(p, "gamma"),
                          vec(p, "beta"), params["str1"]["W"])

    p = params["str1"]
    P = _propagate(g, pair)
    (x_str,) = _tc_sigh_none(degp, P, h, vec(p, "b"), vec(p, "gamma"),
                             vec(p, "beta"))

    return (x_str[:_N], x_att[:_N], x_enc[:_N])
